# baseline (speedup 1.0000x reference)
import jax
import jax.numpy as jnp
from jax import lax
from jax.experimental import pallas as pl
from jax.experimental.pallas import tpu as pltpu
from jax.experimental.pallas import tpu_sc as plsc

N = 10000
E = 320000
D = 128
R = 8
H = 8
C = 64
NC = 2
NS = 16
L = 16
ROWS_PER_TILE = N // NS
NCHUNK = E // C
INV_SQRT_AVG = 0.1767766952966369


def _sc_body(nodes, ef, rad, snd, rcv, w0, w1, zrows, out,
             acc, w0_v, w1_v, send_v, recv_v, rad_v, h_v, msg_v, ef_v):
    c = lax.axis_index("c")
    s = lax.axis_index("s")

    pltpu.sync_copy(zrows, acc.at[pl.ds(s * ROWS_PER_TILE, ROWS_PER_TILE)])
    pltpu.sync_copy(w0, w0_v)
    pltpu.sync_copy(w1, w1_v)
    plsc.subcore_barrier()

    w0vecs = [w0_v[pl.ds(16 * t, L)] for t in range(R * H // L)]
    w0s = [[w0vecs[(i * H + j) // L][(i * H + j) % L] for j in range(H)]
           for i in range(R)]
    coff = c * D
    w1v = [[w1_v[j, pl.ds(coff + 16 * k, L)] for k in range(D // L)]
           for j in range(H)]

    base = NCHUNK // NS
    rem = NCHUNK - base * NS
    nch = base + jnp.where(s < rem, 1, 0)

    def chunk_body(jc, _):
        ch = s + NS * jc
        e0 = ch * C
        pltpu.sync_copy(snd.at[pl.ds(e0, C)], send_v)
        pltpu.sync_copy(rcv.at[pl.ds(e0, C)], recv_v)
        pltpu.sync_copy(rad.at[pl.ds(e0, C)], rad_v)
        pltpu.sync_copy(nodes.at[send_v], msg_v)

        @pl.when(c == 1)
        def _():
            pltpu.sync_copy(ef.at[pl.ds(e0, C)], ef_v)

        def h_body(g, _):
            rows = jnp.int32(L) * g + lax.iota(jnp.int32, L)
            r = [plsc.load_gather(rad_v, [rows, jnp.full((L,), i, jnp.int32)])
                 for i in range(R)]
            for j in range(H):
                a = r[0] * w0s[0][j]
                for i in range(1, R):
                    a = a + r[i] * w0s[i][j]
                h_v[j, pl.ds(L * g, L)] = a / (1.0 + jnp.exp(-a))
            return 0

        lax.fori_loop(0, C // L, h_body, 0)

        def h_bcast(e):
            ve = jnp.full((L,), 0, jnp.int32) + e
            return [plsc.load_gather(h_v, [jnp.full((L,), j, jnp.int32), ve])
                    for j in range(H)]

        def edge0(e, _):
            hs = h_bcast(e)
            for k in range(D // L):
                w = w1v[0][k] * hs[0]
                for j in range(1, H):
                    w = w + w1v[j][k] * hs[j]
                msg_v[e, pl.ds(16 * k, L)] = msg_v[e, pl.ds(16 * k, L)] * w
            return 0

        def edge1(e, _):
            hs = h_bcast(e)
            for k in range(D // L):
                w = w1v[0][k] * hs[0]
                for j in range(1, H):
                    w = w + w1v[j][k] * hs[j]
                msg_v[e, pl.ds(16 * k, L)] = (
                    msg_v[e, pl.ds(16 * k, L)] * ef_v[e, pl.ds(16 * k, L)] * w)
            return 0

        @pl.when(c == 0)
        def _():
            lax.fori_loop(0, C, edge0, 0)

        @pl.when(c == 1)
        def _():
            lax.fori_loop(0, C, edge1, 0)

        pltpu.sync_copy(msg_v, acc.at[recv_v], add=True)
        return 0

    lax.fori_loop(0, nch, chunk_body, 0)

    plsc.subcore_barrier()

    @pl.when(s == 0)
    def _():
        pltpu.sync_copy(acc, out.at[c])


def kernel(node_feats, edge_features, radial_embedding, senders, receivers, W0, W1):
    w0f = W0.reshape(-1).astype(jnp.float32)
    w1s = (W1 * INV_SQRT_AVG).astype(jnp.float32)
    zrows = jnp.zeros((ROWS_PER_TILE, D), jnp.float32)
    mesh = plsc.VectorSubcoreMesh(core_axis_name="c", subcore_axis_name="s")
    f = pl.kernel(
        _sc_body,
        out_type=jax.ShapeDtypeStruct((NC, N, D), jnp.float32),
        mesh=mesh,
        compiler_params=pltpu.CompilerParams(needs_layout_passes=False),
        scratch_types=[
            pltpu.VMEM_SHARED((N, D), jnp.float32),
            pltpu.VMEM((R * H,), jnp.float32),
            pltpu.VMEM((H, 2 * D), jnp.float32),
            pltpu.VMEM((C,), jnp.int32),
            pltpu.VMEM((C,), jnp.int32),
            pltpu.VMEM((C, R), jnp.float32),
            pltpu.VMEM((H, C), jnp.float32),
            pltpu.VMEM((C, D), jnp.float32),
            pltpu.VMEM((C, D), jnp.float32),
        ],
    )
    out2 = f(node_feats, edge_features, radial_embedding, senders, receivers,
             w0f, w1s, zrows)
    return jnp.concatenate([out2[0], out2[1]], axis=-1)


# speedup vs baseline: 1.3052x; 1.3052x over previous
import jax
import jax.numpy as jnp
from jax import lax
from jax.experimental import pallas as pl
from jax.experimental.pallas import tpu as pltpu
from jax.experimental.pallas import tpu_sc as plsc

N = 10000
E = 320000
D = 128
R = 8
H = 8
C = 40
CP = 48
NC = 2
NS = 16
L = 16
ROWS_PER_TILE = N // NS
NCHUNK = E // C
KPS = NCHUNK // NS
INV_SQRT_AVG = 0.1767766952966369


def _sc_body(nodes, ef, rad, snd, rcv, w0, w1, zrows, out,
             acc, w0_v, w1_v,
             send0, send1, recv0, recv1, rad0, rad1, h_v,
             msg0, msg1, ef0, ef1,
             si0, si1, ri0, ri1, g0, g1, s0, s1):
    c = lax.axis_index("c")
    sid = lax.axis_index("s")

    SEND, RECV, RAD = [send0, send1], [recv0, recv1], [rad0, rad1]
    MSG, EF = [msg0, msg1], [ef0, ef1]
    SI, RI, G, S = [si0, si1], [ri0, ri1], [g0, g1], [s0, s1]

    pltpu.sync_copy(zrows, acc.at[pl.ds(sid * ROWS_PER_TILE, ROWS_PER_TILE)])
    pltpu.sync_copy(w0, w0_v)
    pltpu.sync_copy(w1, w1_v)
    plsc.subcore_barrier()

    w0vecs = [w0_v[pl.ds(16 * t, L)] for t in range(R * H // L)]
    w0s = [[w0vecs[(i * H + j) // L][(i * H + j) % L] for j in range(H)]
           for i in range(R)]
    coff = c * D
    w1v = [[w1_v[j, pl.ds(coff + 16 * k, L)] for k in range(D // L)]
           for j in range(H)]

    def e0_of(k):
        return (sid + NS * k) * C

    def issue_in2(k, B):
        e0 = e0_of(k)
        pltpu.async_copy(snd.at[pl.ds(e0, C)], SEND[B], SI[B])
        pltpu.async_copy(rad.at[pl.ds(e0, C)], RAD[B].at[pl.ds(0, C)], SI[B])

        @pl.when(c == 1)
        def _():
            pltpu.async_copy(ef.at[pl.ds(e0, C)], EF[B], SI[B])

    def wait_in2(k, B):
        e0 = e0_of(k)
        pltpu.make_async_copy(snd.at[pl.ds(e0, C)], SEND[B], SI[B]).wait()
        pltpu.make_async_copy(rad.at[pl.ds(e0, C)], RAD[B].at[pl.ds(0, C)],
                              SI[B]).wait()

        @pl.when(c == 1)
        def _():
            pltpu.make_async_copy(ef.at[pl.ds(e0, C)], EF[B], SI[B]).wait()

    def issue_recv(k, B):
        pltpu.async_copy(rcv.at[pl.ds(e0_of(k), C)], RECV[B], RI[B])

    def wait_recv(k, B):
        pltpu.make_async_copy(rcv.at[pl.ds(e0_of(k), C)], RECV[B],
                              RI[B]).wait()

    def issue_gather(B):
        pltpu.async_copy(nodes.at[SEND[B]], MSG[B], G[B])

    def wait_gather(B):
        pltpu.make_async_copy(nodes.at[SEND[B]], MSG[B], G[B]).wait()

    def issue_scatter(B):
        pltpu.async_copy(MSG[B], acc.at[RECV[B]], S[B], add=True)

    def wait_scatter(B):
        pltpu.make_async_copy(MSG[B], acc.at[RECV[B]], S[B]).wait()

    def compute(B):
        msg_v, ef_v, rad_v = MSG[B], EF[B], RAD[B]

        def h_body(g, _):
            rows = jnp.int32(L) * g + lax.iota(jnp.int32, L)
            r = [plsc.load_gather(rad_v, [rows, jnp.full((L,), i, jnp.int32)])
                 for i in range(R)]
            for j in range(H):
                a = r[0] * w0s[0][j]
                for i in range(1, R):
                    a = a + r[i] * w0s[i][j]
                h_v[j, pl.ds(L * g, L)] = a / (1.0 + jnp.exp(-a))
            return 0

        lax.fori_loop(0, CP // L, h_body, 0)

        def h_bcast(e):
            ve = jnp.full((L,), 0, jnp.int32) + e
            return [plsc.load_gather(h_v, [jnp.full((L,), j, jnp.int32), ve])
                    for j in range(H)]

        def edge0(e, _):
            hs = h_bcast(e)
            for k in range(D // L):
                w = w1v[0][k] * hs[0]
                for j in range(1, H):
                    w = w + w1v[j][k] * hs[j]
                msg_v[e, pl.ds(16 * k, L)] = msg_v[e, pl.ds(16 * k, L)] * w
            return 0

        def edge1(e, _):
            hs = h_bcast(e)
            for k in range(D // L):
                w = w1v[0][k] * hs[0]
                for j in range(1, H):
                    w = w + w1v[j][k] * hs[j]
                msg_v[e, pl.ds(16 * k, L)] = (
                    msg_v[e, pl.ds(16 * k, L)] * ef_v[e, pl.ds(16 * k, L)] * w)
            return 0

        @pl.when(c == 0)
        def _():
            lax.fori_loop(0, C, edge0, 0)

        @pl.when(c == 1)
        def _():
            lax.fori_loop(0, C, edge1, 0)

    def half(k, B):
        o = 1 - B
        wait_gather(B)
        compute(B)
        wait_recv(k, B)
        issue_scatter(B)

        @pl.when(k + 2 < KPS)
        def _():
            issue_in2(k + 2, B)

        @pl.when(k + 1 < KPS)
        def _():
            wait_in2(k + 1, o)

            @pl.when(k >= 1)
            def _():
                wait_scatter(o)
                issue_recv(k + 1, o)

            issue_gather(o)

        return 0

    issue_in2(0, 0)
    issue_recv(0, 0)
    issue_in2(1, 1)
    issue_recv(1, 1)
    wait_in2(0, 0)
    issue_gather(0)

    def pair_body(p, _):
        half(2 * p, 0)
        half(2 * p + 1, 1)
        return 0

    lax.fori_loop(0, KPS // 2, pair_body, 0)

    wait_scatter(0)
    wait_scatter(1)
    plsc.subcore_barrier()

    @pl.when(sid == 0)
    def _():
        pltpu.sync_copy(acc, out.at[c])


def kernel(node_feats, edge_features, radial_embedding, senders, receivers, W0, W1):
    w0f = W0.reshape(-1).astype(jnp.float32)
    w1s = (W1 * INV_SQRT_AVG).astype(jnp.float32)
    zrows = jnp.zeros((ROWS_PER_TILE, D), jnp.float32)
    mesh = plsc.VectorSubcoreMesh(core_axis_name="c", subcore_axis_name="s")
    f = pl.kernel(
        _sc_body,
        out_type=jax.ShapeDtypeStruct((NC, N, D), jnp.float32),
        mesh=mesh,
        compiler_params=pltpu.CompilerParams(needs_layout_passes=False),
        scratch_types=[
            pltpu.VMEM_SHARED((N, D), jnp.float32),
            pltpu.VMEM((R * H,), jnp.float32),
            pltpu.VMEM((H, 2 * D), jnp.float32),
            pltpu.VMEM((C,), jnp.int32),
            pltpu.VMEM((C,), jnp.int32),
            pltpu.VMEM((C,), jnp.int32),
            pltpu.VMEM((C,), jnp.int32),
            pltpu.VMEM((CP, R), jnp.float32),
            pltpu.VMEM((CP, R), jnp.float32),
            pltpu.VMEM((H, CP), jnp.float32),
            pltpu.VMEM((C, D), jnp.float32),
            pltpu.VMEM((C, D), jnp.float32),
            pltpu.VMEM((C, D), jnp.float32),
            pltpu.VMEM((C, D), jnp.float32),
            pltpu.SemaphoreType.DMA,
            pltpu.SemaphoreType.DMA,
            pltpu.SemaphoreType.DMA,
            pltpu.SemaphoreType.DMA,
            pltpu.SemaphoreType.DMA,
            pltpu.SemaphoreType.DMA,
            pltpu.SemaphoreType.DMA,
            pltpu.SemaphoreType.DMA,
        ],
    )
    out2 = f(node_feats, edge_features, radial_embedding, senders, receivers,
             w0f, w1s, zrows)
    return jnp.concatenate([out2[0], out2[1]], axis=-1)


# speedup vs baseline: 1.6341x; 1.2520x over previous
import jax
import jax.numpy as jnp
from jax import lax
from jax.experimental import pallas as pl
from jax.experimental.pallas import tpu as pltpu
from jax.experimental.pallas import tpu_sc as plsc

N = 10000
E = 320000
D = 128
DH = 64
R = 8
H = 8
C = 40
CP = 48
NC = 2
NS = 16
L = 16
ROWS_PER_TILE = N // NS
NCHUNK = E // C
KPS = NCHUNK // NS
INV_SQRT_AVG = 0.1767766952966369


def _sc_body(nodes2, ef2, rad, snd, rcv, w0, w1, zrows, out,
             acc, w0_v, w1_v,
             send0, send1, eidx0, eidx1, recv0, recv1, rad0, rad1, h_v,
             msg0, msg1, ef0, ef1, outv0, outv1,
             si0, si1, ri0, ri1, g0, g1, s0, s1):
    c = lax.axis_index("c")
    sid = lax.axis_index("s")

    SEND, EIDX, RECV, RAD = [send0, send1], [eidx0, eidx1], [recv0, recv1], [rad0, rad1]
    MSG, EF, OUTV = [msg0, msg1], [ef0, ef1], [outv0, outv1]
    SI, RI, G, S = [si0, si1], [ri0, ri1], [g0, g1], [s0, s1]

    pltpu.sync_copy(zrows, acc.at[pl.ds(sid * ROWS_PER_TILE, ROWS_PER_TILE)])
    pltpu.sync_copy(w0, w0_v)
    pltpu.sync_copy(w1, w1_v)
    plsc.subcore_barrier()

    w0vecs = [w0_v[pl.ds(16 * t, L)] for t in range(R * H // L)]
    w0s = [[w0vecs[(i * H + j) // L][(i * H + j) % L] for j in range(H)]
           for i in range(R)]
    cm = c * DH
    ct = D + c * DH

    def e0_of(k):
        return (sid + NS * k) * C

    def issue_in2(k, B):
        e0 = e0_of(k)
        pltpu.async_copy(snd.at[pl.ds(e0, C)], SEND[B], SI[B])
        pltpu.async_copy(rad.at[pl.ds(e0, C)], RAD[B].at[pl.ds(0, C)], SI[B])

    def wait_in2(k, B):
        e0 = e0_of(k)
        pltpu.make_async_copy(snd.at[pl.ds(e0, C)], SEND[B], SI[B]).wait()
        pltpu.make_async_copy(rad.at[pl.ds(e0, C)], RAD[B].at[pl.ds(0, C)],
                              SI[B]).wait()

    def issue_recv(k, B):
        pltpu.async_copy(rcv.at[pl.ds(e0_of(k), C)], RECV[B], RI[B])

    def wait_recv(k, B):
        pltpu.make_async_copy(rcv.at[pl.ds(e0_of(k), C)], RECV[B],
                              RI[B]).wait()

    def make_indices(k, B):
        sv = SEND[B]
        a0 = sv[pl.ds(0, L)]
        a1 = sv[pl.ds(16, L)]
        a2 = sv[pl.ds(24, L)]
        sv[pl.ds(0, L)] = a0 * 2 + c
        sv[pl.ds(16, L)] = a1 * 2 + c
        sv[pl.ds(24, L)] = a2 * 2 + c
        ev = EIDX[B]
        base = 2 * e0_of(k) + c
        io = lax.iota(jnp.int32, L) * 2
        ev[pl.ds(0, L)] = io + base
        ev[pl.ds(16, L)] = io + (base + 32)
        ev[pl.ds(24, L)] = io + (base + 48)

    def issue_gathers(B):
        pltpu.async_copy(nodes2.at[SEND[B]], MSG[B], G[B])
        pltpu.async_copy(ef2.at[EIDX[B]], EF[B], G[B])

    def wait_gathers(B):
        pltpu.make_async_copy(nodes2.at[SEND[B]], MSG[B], G[B]).wait()
        pltpu.make_async_copy(ef2.at[EIDX[B]], EF[B], G[B]).wait()

    def issue_scatter(B):
        pltpu.async_copy(OUTV[B], acc.at[RECV[B]], S[B], add=True)

    def wait_scatter(B):
        pltpu.make_async_copy(OUTV[B], acc.at[RECV[B]], S[B]).wait()

    def compute(B):
        msg_v, ef_v, rad_v, out_v = MSG[B], EF[B], RAD[B], OUTV[B]

        def h_body(g, _):
            rows = jnp.int32(L) * g + lax.iota(jnp.int32, L)
            r = [plsc.load_gather(rad_v, [rows, jnp.full((L,), i, jnp.int32)])
                 for i in range(R)]
            for j in range(H):
                a = r[0] * w0s[0][j]
                for i in range(1, R):
                    a = a + r[i] * w0s[i][j]
                h_v[j, pl.ds(L * g, L)] = a / (1.0 + jnp.exp(-a))
            return 0

        lax.fori_loop(0, CP // L, h_body, 0)

        def h_bcast(e):
            ve = jnp.full((L,), 0, jnp.int32) + e
            return [plsc.load_gather(h_v, [jnp.full((L,), j, jnp.int32), ve])
                    for j in range(H)]

        def wsum(wv, k, hs):
            p = [wv[j][k] * hs[j] for j in range(H)]
            q = [p[0] + p[1], p[2] + p[3], p[4] + p[5], p[6] + p[7]]
            return (q[0] + q[1]) + (q[2] + q[3])

        w1m = [[w1_v[j, pl.ds(cm + 16 * k, L)] for k in range(DH // L)]
               for j in range(H)]

        def edge_a(e, _):
            hs = h_bcast(e)
            for k in range(DH // L):
                w = wsum(w1m, k, hs)
                out_v[e, pl.ds(16 * k, L)] = msg_v[e, pl.ds(16 * k, L)] * w
            return 0

        lax.fori_loop(0, C, edge_a, 0)

        w1t = [[w1_v[j, pl.ds(ct + 16 * k, L)] for k in range(DH // L)]
               for j in range(H)]

        def edge_b(e, _):
            hs = h_bcast(e)
            for k in range(DH // L):
                w = wsum(w1t, k, hs)
                out_v[e, pl.ds(DH + 16 * k, L)] = (
                    msg_v[e, pl.ds(16 * k, L)] * ef_v[e, pl.ds(16 * k, L)] * w)
            return 0

        lax.fori_loop(0, C, edge_b, 0)

    def half(k, B):
        o = 1 - B
        wait_gathers(B)
        compute(B)
        wait_recv(k, B)
        issue_scatter(B)

        @pl.when(k + 2 < KPS)
        def _():
            issue_in2(k + 2, B)

        @pl.when(k + 1 < KPS)
        def _():
            wait_in2(k + 1, o)

            @pl.when(k >= 1)
            def _():
                wait_scatter(o)
                issue_recv(k + 1, o)

            make_indices(k + 1, o)
            issue_gathers(o)

        return 0

    issue_in2(0, 0)
    issue_recv(0, 0)
    issue_in2(1, 1)
    issue_recv(1, 1)
    wait_in2(0, 0)
    make_indices(0, 0)
    issue_gathers(0)

    def pair_body(p, _):
        half(2 * p, 0)
        half(2 * p + 1, 1)
        return 0

    lax.fori_loop(0, KPS // 2, pair_body, 0)

    wait_scatter(0)
    wait_scatter(1)
    plsc.subcore_barrier()

    @pl.when(sid == 0)
    def _():
        pltpu.sync_copy(acc, out.at[c])


def kernel(node_feats, edge_features, radial_embedding, senders, receivers, W0, W1):
    nodes2 = node_feats.reshape(2 * N, DH)
    ef2 = edge_features.reshape(2 * E, DH)
    w0f = W0.reshape(-1).astype(jnp.float32)
    w1s = (W1 * INV_SQRT_AVG).astype(jnp.float32)
    zrows = jnp.zeros((ROWS_PER_TILE, D), jnp.float32)
    mesh = plsc.VectorSubcoreMesh(core_axis_name="c", subcore_axis_name="s")
    f = pl.kernel(
        _sc_body,
        out_type=jax.ShapeDtypeStruct((NC, N, D), jnp.float32),
        mesh=mesh,
        compiler_params=pltpu.CompilerParams(needs_layout_passes=False,
                                             use_tc_tiling_on_sc=False),
        scratch_types=[
            pltpu.VMEM_SHARED((N, D), jnp.float32),
            pltpu.VMEM((R * H,), jnp.float32),
            pltpu.VMEM((H, 2 * D), jnp.float32),
            pltpu.VMEM((C,), jnp.int32),
            pltpu.VMEM((C,), jnp.int32),
            pltpu.VMEM((C,), jnp.int32),
            pltpu.VMEM((C,), jnp.int32),
            pltpu.VMEM((C,), jnp.int32),
            pltpu.VMEM((C,), jnp.int32),
            pltpu.VMEM((CP, R), jnp.float32),
            pltpu.VMEM((CP, R), jnp.float32),
            pltpu.VMEM((H, CP), jnp.float32),
            pltpu.VMEM((C, DH), jnp.float32),
            pltpu.VMEM((C, DH), jnp.float32),
            pltpu.VMEM((C, DH), jnp.float32),
            pltpu.VMEM((C, DH), jnp.float32),
            pltpu.VMEM((C, D), jnp.float32),
            pltpu.VMEM((C, D), jnp.float32),
            pltpu.SemaphoreType.DMA,
            pltpu.SemaphoreType.DMA,
            pltpu.SemaphoreType.DMA,
            pltpu.SemaphoreType.DMA,
            pltpu.SemaphoreType.DMA,
            pltpu.SemaphoreType.DMA,
            pltpu.SemaphoreType.DMA,
            pltpu.SemaphoreType.DMA,
        ],
    )
    out2 = f(nodes2, ef2, radial_embedding, senders, receivers, w0f, w1s, zrows)
    return jnp.concatenate(
        [out2[0, :, :DH], out2[1, :, :DH], out2[0, :, DH:], out2[1, :, DH:]],
        axis=-1)


# speedup vs baseline: 2.2986x; 1.4067x over previous
import jax
import jax.numpy as jnp
from jax import lax
from jax.experimental import pallas as pl
from jax.experimental.pallas import tpu as pltpu
from jax.experimental.pallas import tpu_sc as plsc

N = 10000
E = 320000
D = 128
DH = 64
R = 8
H = 8
C = 40
CP = 48
NC = 2
NS = 16
L = 16
ROWS_PER_TILE = N // NS
NCHUNK = E // C
KPS = NCHUNK // NS
INV_SQRT_AVG = 0.1767766952966369


def _sc_body(nodes2, ef2, rad, snd, rcv, w0, w1, zrows, out,
             acc, w0_v, w1_v,
             send0, send1, eidx0, eidx1, recv0, recv1, rad0, rad1, h_v,
             msg0, msg1, ef0, ef1, outv0, outv1,
             si0, si1, ri0, ri1, g0, g1, s0, s1):
    c = lax.axis_index("c")
    sid = lax.axis_index("s")

    SEND, EIDX, RECV, RAD = [send0, send1], [eidx0, eidx1], [recv0, recv1], [rad0, rad1]
    MSG, EF, OUTV = [msg0, msg1], [ef0, ef1], [outv0, outv1]
    SI, RI, G, S = [si0, si1], [ri0, ri1], [g0, g1], [s0, s1]

    pltpu.sync_copy(zrows, acc.at[pl.ds(sid * ROWS_PER_TILE, ROWS_PER_TILE)])
    pltpu.sync_copy(w0, w0_v)
    pltpu.sync_copy(w1, w1_v)
    plsc.subcore_barrier()

    w0vecs = [w0_v[pl.ds(16 * t, L)] for t in range(R * H // L)]
    w0s = [[w0vecs[(i * H + j) // L][(i * H + j) % L] for j in range(H)]
           for i in range(R)]
    cm = c * DH
    ct = D + c * DH

    def e0_of(k):
        return (sid + NS * k) * C

    def issue_in2(k, B):
        e0 = e0_of(k)
        pltpu.async_copy(snd.at[pl.ds(e0, C)], SEND[B], SI[B])
        pltpu.async_copy(rad.at[pl.ds(e0, C)], RAD[B].at[pl.ds(0, C)], SI[B])

    def wait_in2(k, B):
        e0 = e0_of(k)
        pltpu.make_async_copy(snd.at[pl.ds(e0, C)], SEND[B], SI[B]).wait()
        pltpu.make_async_copy(rad.at[pl.ds(e0, C)], RAD[B].at[pl.ds(0, C)],
                              SI[B]).wait()

    def issue_recv(k, B):
        pltpu.async_copy(rcv.at[pl.ds(e0_of(k), C)], RECV[B], RI[B])

    def wait_recv(k, B):
        pltpu.make_async_copy(rcv.at[pl.ds(e0_of(k), C)], RECV[B],
                              RI[B]).wait()

    def make_indices(k, B):
        sv = SEND[B]
        a0 = sv[pl.ds(0, L)]
        a1 = sv[pl.ds(16, L)]
        a2 = sv[pl.ds(24, L)]
        sv[pl.ds(0, L)] = a0 * 2 + c
        sv[pl.ds(16, L)] = a1 * 2 + c
        sv[pl.ds(24, L)] = a2 * 2 + c
        ev = EIDX[B]
        base = 2 * e0_of(k) + c
        io = lax.iota(jnp.int32, L) * 2
        ev[pl.ds(0, L)] = io + base
        ev[pl.ds(16, L)] = io + (base + 32)
        ev[pl.ds(24, L)] = io + (base + 48)

    def issue_gathers(B):
        pltpu.async_copy(nodes2.at[SEND[B]], MSG[B], G[B])
        pltpu.async_copy(ef2.at[EIDX[B]], EF[B], G[B])

    def wait_gathers(B):
        pltpu.make_async_copy(nodes2.at[SEND[B]], MSG[B], G[B]).wait()
        pltpu.make_async_copy(ef2.at[EIDX[B]], EF[B], G[B]).wait()

    def issue_scatter(B):
        pltpu.async_copy(OUTV[B], acc.at[RECV[B]], S[B], add=True)

    def wait_scatter(B):
        pltpu.make_async_copy(OUTV[B], acc.at[RECV[B]], S[B]).wait()

    def compute(B):
        msg_v, ef_v, rad_v, out_v = MSG[B], EF[B], RAD[B], OUTV[B]

        @plsc.parallel_loop(0, CP // L)
        def _(g):
            rows = jnp.int32(L) * g + lax.iota(jnp.int32, L)
            r = [plsc.load_gather(rad_v, [rows, jnp.full((L,), i, jnp.int32)])
                 for i in range(R)]
            for j in range(H):
                a = r[0] * w0s[0][j]
                for i in range(1, R):
                    a = a + r[i] * w0s[i][j]
                h_v[j, pl.ds(L * g, L)] = a / (1.0 + jnp.exp(-a))

        def h_bcast(e):
            ve = jnp.full((L,), 0, jnp.int32) + e
            return [plsc.load_gather(h_v, [jnp.full((L,), j, jnp.int32), ve])
                    for j in range(H)]

        def wsum(wv, k, hs):
            p = [wv[j][k] * hs[j] for j in range(H)]
            q = [p[0] + p[1], p[2] + p[3], p[4] + p[5], p[6] + p[7]]
            return (q[0] + q[1]) + (q[2] + q[3])

        w1m = [[w1_v[j, pl.ds(cm + 16 * k, L)] for k in range(DH // L)]
               for j in range(H)]

        @plsc.parallel_loop(0, C)
        def _(e):
            hs = h_bcast(e)
            for k in range(DH // L):
                w = wsum(w1m, k, hs)
                out_v[e, pl.ds(16 * k, L)] = msg_v[e, pl.ds(16 * k, L)] * w

        w1t = [[w1_v[j, pl.ds(ct + 16 * k, L)] for k in range(DH // L)]
               for j in range(H)]

        @plsc.parallel_loop(0, C)
        def _(e):
            hs = h_bcast(e)
            for k in range(DH // L):
                w = wsum(w1t, k, hs)
                out_v[e, pl.ds(DH + 16 * k, L)] = (
                    msg_v[e, pl.ds(16 * k, L)] * ef_v[e, pl.ds(16 * k, L)] * w)

    def half(k, B):
        o = 1 - B
        wait_gathers(B)
        compute(B)
        wait_recv(k, B)
        issue_scatter(B)

        @pl.when(k + 2 < KPS)
        def _():
            issue_in2(k + 2, B)

        @pl.when(k + 1 < KPS)
        def _():
            wait_in2(k + 1, o)

            @pl.when(k >= 1)
            def _():
                wait_scatter(o)
                issue_recv(k + 1, o)

            make_indices(k + 1, o)
            issue_gathers(o)

        return 0

    issue_in2(0, 0)
    issue_recv(0, 0)
    issue_in2(1, 1)
    issue_recv(1, 1)
    wait_in2(0, 0)
    make_indices(0, 0)
    issue_gathers(0)

    def pair_body(p, _):
        half(2 * p, 0)
        half(2 * p + 1, 1)
        return 0

    lax.fori_loop(0, KPS // 2, pair_body, 0)

    wait_scatter(0)
    wait_scatter(1)
    plsc.subcore_barrier()

    @pl.when(sid == 0)
    def _():
        pltpu.sync_copy(acc, out.at[c])


def kernel(node_feats, edge_features, radial_embedding, senders, receivers, W0, W1):
    nodes2 = node_feats.reshape(2 * N, DH)
    ef2 = edge_features.reshape(2 * E, DH)
    w0f = W0.reshape(-1).astype(jnp.float32)
    w1s = (W1 * INV_SQRT_AVG).astype(jnp.float32)
    zrows = jnp.zeros((ROWS_PER_TILE, D), jnp.float32)
    mesh = plsc.VectorSubcoreMesh(core_axis_name="c", subcore_axis_name="s")
    f = pl.kernel(
        _sc_body,
        out_type=jax.ShapeDtypeStruct((NC, N, D), jnp.float32),
        mesh=mesh,
        compiler_params=pltpu.CompilerParams(needs_layout_passes=False,
                                             use_tc_tiling_on_sc=False),
        scratch_types=[
            pltpu.VMEM_SHARED((N, D), jnp.float32),
            pltpu.VMEM((R * H,), jnp.float32),
            pltpu.VMEM((H, 2 * D), jnp.float32),
            pltpu.VMEM((C,), jnp.int32),
            pltpu.VMEM((C,), jnp.int32),
            pltpu.VMEM((C,), jnp.int32),
            pltpu.VMEM((C,), jnp.int32),
            pltpu.VMEM((C,), jnp.int32),
            pltpu.VMEM((C,), jnp.int32),
            pltpu.VMEM((CP, R), jnp.float32),
            pltpu.VMEM((CP, R), jnp.float32),
            pltpu.VMEM((H, CP), jnp.float32),
            pltpu.VMEM((C, DH), jnp.float32),
            pltpu.VMEM((C, DH), jnp.float32),
            pltpu.VMEM((C, DH), jnp.float32),
            pltpu.VMEM((C, DH), jnp.float32),
            pltpu.VMEM((C, D), jnp.float32),
            pltpu.VMEM((C, D), jnp.float32),
            pltpu.SemaphoreType.DMA,
            pltpu.SemaphoreType.DMA,
            pltpu.SemaphoreType.DMA,
            pltpu.SemaphoreType.DMA,
            pltpu.SemaphoreType.DMA,
            pltpu.SemaphoreType.DMA,
            pltpu.SemaphoreType.DMA,
            pltpu.SemaphoreType.DMA,
        ],
    )
    out2 = f(nodes2, ef2, radial_embedding, senders, receivers, w0f, w1s, zrows)
    return jnp.concatenate(
        [out2[0, :, :DH], out2[1, :, :DH], out2[0, :, DH:], out2[1, :, DH:]],
        axis=-1)


# speedup vs baseline: 2.5201x; 1.0964x over previous
import jax
import jax.numpy as jnp
from jax import lax
from jax.experimental import pallas as pl
from jax.experimental.pallas import tpu as pltpu
from jax.experimental.pallas import tpu_sc as plsc

N = 10000
E = 320000
D = 128
DH = 64
R = 8
H = 8
C = 80
CP = 80
NC = 2
NS = 16
L = 16
ROWS_PER_TILE = N // NS
NCHUNK = E // C
KPS = NCHUNK // NS
INV_SQRT_AVG = 0.1767766952966369


def _sc_body(nodes2, ef2, rad, snd, rcv, w0, w1, zrows, out,
             acc_m, acc_t, w0_v, w1_v,
             send0, send1, eidx0, eidx1, recv0, recv1, rad0, rad1, h_v,
             msg0, msg1, ef0, ef1,
             si0, si1, ri0, ri1, g0, g1, s0, s1):
    c = lax.axis_index("c")
    sid = lax.axis_index("s")

    SEND, EIDX, RECV, RAD = [send0, send1], [eidx0, eidx1], [recv0, recv1], [rad0, rad1]
    MSG, EF = [msg0, msg1], [ef0, ef1]
    SI, RI, G, S = [si0, si1], [ri0, ri1], [g0, g1], [s0, s1]

    pltpu.sync_copy(zrows, acc_m.at[pl.ds(sid * ROWS_PER_TILE, ROWS_PER_TILE)])
    pltpu.sync_copy(zrows, acc_t.at[pl.ds(sid * ROWS_PER_TILE, ROWS_PER_TILE)])
    pltpu.sync_copy(w0, w0_v)
    pltpu.sync_copy(w1, w1_v)
    plsc.subcore_barrier()

    w0vecs = [w0_v[pl.ds(16 * t, L)] for t in range(R * H // L)]
    w0s = [[w0vecs[(i * H + j) // L][(i * H + j) % L] for j in range(H)]
           for i in range(R)]
    cm = c * DH
    ct = D + c * DH

    def e0_of(k):
        return (sid + NS * k) * C

    def issue_in2(k, B):
        e0 = e0_of(k)
        pltpu.async_copy(snd.at[pl.ds(e0, C)], SEND[B], SI[B])
        pltpu.async_copy(rad.at[pl.ds(e0, C)], RAD[B].at[pl.ds(0, C)], SI[B])

    def wait_in2(k, B):
        e0 = e0_of(k)
        pltpu.make_async_copy(snd.at[pl.ds(e0, C)], SEND[B], SI[B]).wait()
        pltpu.make_async_copy(rad.at[pl.ds(e0, C)], RAD[B].at[pl.ds(0, C)],
                              SI[B]).wait()

    def issue_recv(k, B):
        pltpu.async_copy(rcv.at[pl.ds(e0_of(k), C)], RECV[B], RI[B])

    def wait_recv(k, B):
        pltpu.make_async_copy(rcv.at[pl.ds(e0_of(k), C)], RECV[B],
                              RI[B]).wait()

    def make_indices(k, B):
        sv, ev = SEND[B], EIDX[B]
        base = 2 * e0_of(k) + c
        io = lax.iota(jnp.int32, L) * 2
        for t in range(C // L):
            sv[pl.ds(L * t, L)] = sv[pl.ds(L * t, L)] * 2 + c
            ev[pl.ds(L * t, L)] = io + (base + 2 * L * t)

    def issue_gathers(B):
        pltpu.async_copy(nodes2.at[SEND[B]], MSG[B], G[B])
        pltpu.async_copy(ef2.at[EIDX[B]], EF[B], G[B])

    def wait_gathers(B):
        pltpu.make_async_copy(nodes2.at[SEND[B]], MSG[B], G[B]).wait()
        pltpu.make_async_copy(ef2.at[EIDX[B]], EF[B], G[B]).wait()

    def issue_scatter(B):
        pltpu.async_copy(MSG[B], acc_m.at[RECV[B]], S[B], add=True)
        pltpu.async_copy(EF[B], acc_t.at[RECV[B]], S[B], add=True)

    def wait_scatter(B):
        pltpu.make_async_copy(MSG[B], acc_m.at[RECV[B]], S[B]).wait()
        pltpu.make_async_copy(EF[B], acc_t.at[RECV[B]], S[B]).wait()

    def compute(B):
        msg_v, ef_v, rad_v = MSG[B], EF[B], RAD[B]

        @plsc.parallel_loop(0, CP // L)
        def _(g):
            rows = jnp.int32(L) * g + lax.iota(jnp.int32, L)
            r = [plsc.load_gather(rad_v, [rows, jnp.full((L,), i, jnp.int32)])
                 for i in range(R)]
            for j in range(H):
                a = r[0] * w0s[0][j]
                for i in range(1, R):
                    a = a + r[i] * w0s[i][j]
                h_v[j, pl.ds(L * g, L)] = a / (1.0 + jnp.exp(-a))

        def h_bcast(e):
            ve = jnp.full((L,), 0, jnp.int32) + e
            return [plsc.load_gather(h_v, [jnp.full((L,), j, jnp.int32), ve])
                    for j in range(H)]

        def wsum(wv, k, hs):
            p = [wv[j][k] * hs[j] for j in range(H)]
            q = [p[0] + p[1], p[2] + p[3], p[4] + p[5], p[6] + p[7]]
            return (q[0] + q[1]) + (q[2] + q[3])

        w1t = [[w1_v[j, pl.ds(ct + 16 * k, L)] for k in range(DH // L)]
               for j in range(H)]

        @plsc.parallel_loop(0, C)
        def _(e):
            hs = h_bcast(e)
            for k in range(DH // L):
                w = wsum(w1t, k, hs)
                ef_v[e, pl.ds(16 * k, L)] = (
                    msg_v[e, pl.ds(16 * k, L)] * ef_v[e, pl.ds(16 * k, L)] * w)

        w1m = [[w1_v[j, pl.ds(cm + 16 * k, L)] for k in range(DH // L)]
               for j in range(H)]

        @plsc.parallel_loop(0, C)
        def _(e):
            hs = h_bcast(e)
            for k in range(DH // L):
                w = wsum(w1m, k, hs)
                msg_v[e, pl.ds(16 * k, L)] = msg_v[e, pl.ds(16 * k, L)] * w

    def half(k, B):
        o = 1 - B
        wait_gathers(B)
        compute(B)
        wait_recv(k, B)
        issue_scatter(B)

        @pl.when(k + 2 < KPS)
        def _():
            issue_in2(k + 2, B)

        @pl.when(k + 1 < KPS)
        def _():
            wait_in2(k + 1, o)

            @pl.when(k >= 1)
            def _():
                wait_scatter(o)
                issue_recv(k + 1, o)

            make_indices(k + 1, o)
            issue_gathers(o)

        return 0

    issue_in2(0, 0)
    issue_recv(0, 0)
    issue_in2(1, 1)
    issue_recv(1, 1)
    wait_in2(0, 0)
    make_indices(0, 0)
    issue_gathers(0)

    def pair_body(p, _):
        half(2 * p, 0)
        half(2 * p + 1, 1)
        return 0

    lax.fori_loop(0, KPS // 2, pair_body, 0)

    wait_scatter(0)
    wait_scatter(1)
    plsc.subcore_barrier()

    @pl.when(sid == 0)
    def _():
        pltpu.sync_copy(acc_m, out.at[c, 0])
        pltpu.sync_copy(acc_t, out.at[c, 1])


def kernel(node_feats, edge_features, radial_embedding, senders, receivers, W0, W1):
    nodes2 = node_feats.reshape(2 * N, DH)
    ef2 = edge_features.reshape(2 * E, DH)
    w0f = W0.reshape(-1).astype(jnp.float32)
    w1s = (W1 * INV_SQRT_AVG).astype(jnp.float32)
    zrows = jnp.zeros((ROWS_PER_TILE, DH), jnp.float32)
    mesh = plsc.VectorSubcoreMesh(core_axis_name="c", subcore_axis_name="s")
    f = pl.kernel(
        _sc_body,
        out_type=jax.ShapeDtypeStruct((NC, 2, N, DH), jnp.float32),
        mesh=mesh,
        compiler_params=pltpu.CompilerParams(needs_layout_passes=False,
                                             use_tc_tiling_on_sc=False),
        scratch_types=[
            pltpu.VMEM_SHARED((N, DH), jnp.float32),
            pltpu.VMEM_SHARED((N, DH), jnp.float32),
            pltpu.VMEM((R * H,), jnp.float32),
            pltpu.VMEM((H, 2 * D), jnp.float32),
            pltpu.VMEM((C,), jnp.int32),
            pltpu.VMEM((C,), jnp.int32),
            pltpu.VMEM((C,), jnp.int32),
            pltpu.VMEM((C,), jnp.int32),
            pltpu.VMEM((C,), jnp.int32),
            pltpu.VMEM((C,), jnp.int32),
            pltpu.VMEM((CP, R), jnp.float32),
            pltpu.VMEM((CP, R), jnp.float32),
            pltpu.VMEM((H, CP), jnp.float32),
            pltpu.VMEM((C, DH), jnp.float32),
            pltpu.VMEM((C, DH), jnp.float32),
            pltpu.VMEM((C, DH), jnp.float32),
            pltpu.VMEM((C, DH), jnp.float32),
            pltpu.SemaphoreType.DMA,
            pltpu.SemaphoreType.DMA,
            pltpu.SemaphoreType.DMA,
            pltpu.SemaphoreType.DMA,
            pltpu.SemaphoreType.DMA,
            pltpu.SemaphoreType.DMA,
            pltpu.SemaphoreType.DMA,
            pltpu.SemaphoreType.DMA,
        ],
    )
    out2 = f(nodes2, ef2, radial_embedding, senders, receivers, w0f, w1s, zrows)
    return jnp.concatenate(
        [out2[0, 0], out2[1, 0], out2[0, 1], out2[1, 1]], axis=-1)


# speedup vs baseline: 3.3841x; 1.3428x over previous
import jax
import jax.numpy as jnp
from jax import lax
from jax.experimental import pallas as pl
from jax.experimental.pallas import tpu as pltpu
from jax.experimental.pallas import tpu_sc as plsc

N = 10000
E = 320000
D = 128
DH = 64
R = 8
H = 8
C = 80
CP = 80
NC = 2
NS = 16
L = 16
ROWS_PER_TILE = N // NS
NCHUNK = E // C
KPS = NCHUNK // NS
INV_SQRT_AVG = 0.1767766952966369


def _sc_body(nodes2, ef2, rad, snd, rcv, w0, w1, zrows, out,
             acc_m, acc_t, w0_v, w1_v,
             send0, send1, eidx0, eidx1, recv0, recv1, rad0, rad1, h_v,
             msg0, msg1, ef0, ef1,
             si0, si1, ri0, ri1, g0, g1, s0, s1):
    c = lax.axis_index("c")
    sid = lax.axis_index("s")

    SEND, EIDX, RECV, RAD = [send0, send1], [eidx0, eidx1], [recv0, recv1], [rad0, rad1]
    MSG, EF = [msg0, msg1], [ef0, ef1]
    SI, RI, G, S = [si0, si1], [ri0, ri1], [g0, g1], [s0, s1]

    pltpu.sync_copy(zrows, acc_m.at[pl.ds(sid * ROWS_PER_TILE, ROWS_PER_TILE)])
    pltpu.sync_copy(zrows, acc_t.at[pl.ds(sid * ROWS_PER_TILE, ROWS_PER_TILE)])
    pltpu.sync_copy(w0, w0_v)
    pltpu.sync_copy(w1, w1_v)
    plsc.subcore_barrier()

    w0vecs = [w0_v[pl.ds(16 * t, L)] for t in range(R * H // L)]
    w0s = [[w0vecs[(i * H + j) // L][(i * H + j) % L] for j in range(H)]
           for i in range(R)]
    cm = c * DH
    ct = D + c * DH

    def e0_of(k):
        return (sid + NS * k) * C

    def issue_in2(k, B):
        e0 = e0_of(k)
        pltpu.async_copy(snd.at[pl.ds(e0, C)], SEND[B], SI[B])
        pltpu.async_copy(rad.at[pl.ds(e0, C)], RAD[B].at[pl.ds(0, C)], SI[B])

    def wait_in2(k, B):
        e0 = e0_of(k)
        pltpu.make_async_copy(snd.at[pl.ds(e0, C)], SEND[B], SI[B]).wait()
        pltpu.make_async_copy(rad.at[pl.ds(e0, C)], RAD[B].at[pl.ds(0, C)],
                              SI[B]).wait()

    def issue_recv(k, B):
        pltpu.async_copy(rcv.at[pl.ds(e0_of(k), C)], RECV[B], RI[B])

    def wait_recv(k, B):
        pltpu.make_async_copy(rcv.at[pl.ds(e0_of(k), C)], RECV[B],
                              RI[B]).wait()

    def make_indices(k, B):
        sv, ev = SEND[B], EIDX[B]
        base = 2 * e0_of(k) + c
        io = lax.iota(jnp.int32, L) * 2
        for t in range(C // L):
            sv[pl.ds(L * t, L)] = sv[pl.ds(L * t, L)] * 2 + c
            ev[pl.ds(L * t, L)] = io + (base + 2 * L * t)

    def issue_gathers(B):
        pltpu.async_copy(nodes2.at[SEND[B]], MSG[B], G[B])
        pltpu.async_copy(ef2.at[EIDX[B]], EF[B], G[B])

    def wait_gathers(B):
        pltpu.make_async_copy(nodes2.at[SEND[B]], MSG[B], G[B]).wait()
        pltpu.make_async_copy(ef2.at[EIDX[B]], EF[B], G[B]).wait()

    def issue_scatter(B):
        pltpu.async_copy(MSG[B], acc_m.at[RECV[B]], S[B], add=True)
        pltpu.async_copy(EF[B], acc_t.at[RECV[B]], S[B], add=True)

    def wait_scatter(B):
        pltpu.make_async_copy(MSG[B], acc_m.at[RECV[B]], S[B]).wait()
        pltpu.make_async_copy(EF[B], acc_t.at[RECV[B]], S[B]).wait()

    def compute(B):
        msg_v, ef_v, rad_v = MSG[B], EF[B], RAD[B]

        @plsc.parallel_loop(0, CP // L)
        def _(g):
            rows = jnp.int32(L) * g + lax.iota(jnp.int32, L)
            r = [plsc.load_gather(rad_v, [rows, jnp.full((L,), i, jnp.int32)])
                 for i in range(R)]
            for j in range(H):
                a = r[0] * w0s[0][j]
                for i in range(1, R):
                    a = a + r[i] * w0s[i][j]
                h_v[j, pl.ds(L * g, L)] = a / (1.0 + jnp.exp(-a))

        def h_bcast(e):
            ve = jnp.full((L,), 0, jnp.int32) + e
            return [plsc.load_gather(h_v, [jnp.full((L,), j, jnp.int32), ve])
                    for j in range(H)]

        def wsum_bf(wv, k2, hsbf):
            p = [wv[j][k2] * hsbf[j] for j in range(H)]
            q = [p[0] + p[1], p[2] + p[3], p[4] + p[5], p[6] + p[7]]
            return (q[0] + q[1]) + (q[2] + q[3])

        w1m = [[w1_v[j, pl.ds(cm + 32 * k2, 2 * L)] for k2 in range(DH // (2 * L))]
               for j in range(H)]
        w1t = [[w1_v[j, pl.ds(ct + 32 * k2, 2 * L)] for k2 in range(DH // (2 * L))]
               for j in range(H)]

        @plsc.parallel_loop(0, C)
        def _(e):
            hs = h_bcast(e)
            hsbf = [plsc.pack(hs[j], hs[j], format=plsc.PackFormat.INTERLEAVED)
                    for j in range(H)]
            for k2 in range(DH // (2 * L)):
                lo, hi = pl.ds(32 * k2, L), pl.ds(32 * k2 + L, L)
                m0, m1 = msg_v[e, lo], msg_v[e, hi]
                wt0, wt1 = plsc.unpack(wsum_bf(w1t, k2, hsbf),
                                       format=plsc.PackFormat.INTERLEAVED)
                ef_v[e, lo] = m0 * ef_v[e, lo] * wt0
                ef_v[e, hi] = m1 * ef_v[e, hi] * wt1
                wm0, wm1 = plsc.unpack(wsum_bf(w1m, k2, hsbf),
                                       format=plsc.PackFormat.INTERLEAVED)
                msg_v[e, lo] = m0 * wm0
                msg_v[e, hi] = m1 * wm1

    def half(k, B):
        o = 1 - B
        wait_gathers(B)
        compute(B)
        wait_recv(k, B)
        issue_scatter(B)

        @pl.when(k + 2 < KPS)
        def _():
            issue_in2(k + 2, B)

        @pl.when(k + 1 < KPS)
        def _():
            wait_in2(k + 1, o)

            @pl.when(k >= 1)
            def _():
                wait_scatter(o)
                issue_recv(k + 1, o)

            make_indices(k + 1, o)
            issue_gathers(o)

        return 0

    issue_in2(0, 0)
    issue_recv(0, 0)
    issue_in2(1, 1)
    issue_recv(1, 1)
    wait_in2(0, 0)
    make_indices(0, 0)
    issue_gathers(0)

    def pair_body(p, _):
        half(2 * p, 0)
        half(2 * p + 1, 1)
        return 0

    lax.fori_loop(0, KPS // 2, pair_body, 0)

    wait_scatter(0)
    wait_scatter(1)
    plsc.subcore_barrier()

    @pl.when(sid == 0)
    def _():
        pltpu.sync_copy(acc_m, out.at[c, 0])
        pltpu.sync_copy(acc_t, out.at[c, 1])


def kernel(node_feats, edge_features, radial_embedding, senders, receivers, W0, W1):
    nodes2 = node_feats.reshape(2 * N, DH)
    ef2 = edge_features.reshape(2 * E, DH)
    w0f = W0.reshape(-1).astype(jnp.float32)
    w1s = (W1 * INV_SQRT_AVG).astype(jnp.float32)
    w1s = (w1s.reshape(H, 2 * D // 32, 2, 16)
              .transpose(0, 1, 3, 2)
              .reshape(H, 2 * D)
              .astype(jnp.bfloat16))
    zrows = jnp.zeros((ROWS_PER_TILE, DH), jnp.float32)
    mesh = plsc.VectorSubcoreMesh(core_axis_name="c", subcore_axis_name="s")
    f = pl.kernel(
        _sc_body,
        out_type=jax.ShapeDtypeStruct((NC, 2, N, DH), jnp.float32),
        mesh=mesh,
        compiler_params=pltpu.CompilerParams(needs_layout_passes=False,
                                             use_tc_tiling_on_sc=False),
        scratch_types=[
            pltpu.VMEM_SHARED((N, DH), jnp.float32),
            pltpu.VMEM_SHARED((N, DH), jnp.float32),
            pltpu.VMEM((R * H,), jnp.float32),
            pltpu.VMEM((H, 2 * D), jnp.bfloat16),
            pltpu.VMEM((C,), jnp.int32),
            pltpu.VMEM((C,), jnp.int32),
            pltpu.VMEM((C,), jnp.int32),
            pltpu.VMEM((C,), jnp.int32),
            pltpu.VMEM((C,), jnp.int32),
            pltpu.VMEM((C,), jnp.int32),
            pltpu.VMEM((CP, R), jnp.float32),
            pltpu.VMEM((CP, R), jnp.float32),
            pltpu.VMEM((H, CP), jnp.float32),
            pltpu.VMEM((C, DH), jnp.float32),
            pltpu.VMEM((C, DH), jnp.float32),
            pltpu.VMEM((C, DH), jnp.float32),
            pltpu.VMEM((C, DH), jnp.float32),
            pltpu.SemaphoreType.DMA,
            pltpu.SemaphoreType.DMA,
            pltpu.SemaphoreType.DMA,
            pltpu.SemaphoreType.DMA,
            pltpu.SemaphoreType.DMA,
            pltpu.SemaphoreType.DMA,
            pltpu.SemaphoreType.DMA,
            pltpu.SemaphoreType.DMA,
        ],
    )
    out2 = f(nodes2, ef2, radial_embedding, senders, receivers, w0f, w1s, zrows)
    return jnp.concatenate(
        [out2[0, 0], out2[1, 0], out2[0, 1], out2[1, 1]], axis=-1)
